# Initial kernel scaffold; baseline (speedup 1.0000x reference)
import jax
import jax.numpy as jnp
from jax.experimental import pallas as pl


def kernel(inputs, frames, edge_index, W_in, b_in, W1a_0, b1a_0, W1b_0, b1b_0, W1a_1, b1a_1, W1b_1, b1b_1, W_out, b_out):
    raise NotImplementedError("write your pallas kernel here")


# speedup vs baseline: 4.8575x; 4.8575x over previous
import functools

import jax
import jax.numpy as jnp
from jax import lax
from jax.experimental import pallas as pl
from jax.experimental.pallas import tpu as pltpu
from jax.experimental.pallas import tpu_sc as plsc

N = 10000
E = 640000
D = 128

NC = 2
NS = 16
L = 16
NW = NC * NS
EPW = E // NW
C = 40
NCHUNK = EPW // C
ZROWS = C
NZCHUNK = N // ZROWS
ZITER = -(-NZCHUNK // NS)
G = D // L


_TCB = 1000
_TCG = N // _TCB


def _dot(a, b):
    return jnp.dot(a, b, preferred_element_type=jnp.float32)


def _tc_head_body(x_ref, win_ref, bin_ref, wat_ref, wab_ref, bav_ref,
                  u_ref, v_ref):
    x = _dot(x_ref[...], win_ref[...]) + bin_ref[...]
    u_ref[...] = _dot(x, wat_ref[...])
    v_ref[...] = _dot(x, wab_ref[...]) + bav_ref[...]


def _tc_mid_body(p_ref, deg_ref, wb_ref, bb_ref, wat_ref, wab_ref, bav_ref,
                 u_ref, v_ref):
    s = p_ref[0] + p_ref[1]
    deg = deg_ref[0, :, 0:1] + deg_ref[1, :, 0:1]
    x = _dot(s, wb_ref[...]) + deg * bb_ref[...]
    u_ref[...] = _dot(x, wat_ref[...])
    v_ref[...] = _dot(x, wab_ref[...]) + bav_ref[...]


def _tc_tail_body(p_ref, deg_ref, wb_ref, bb_ref, wout_ref, bout_ref, o_ref):
    s = p_ref[0] + p_ref[1]
    deg = deg_ref[0, :, 0:1] + deg_ref[1, :, 0:1]
    x = _dot(s, wb_ref[...]) + deg * bb_ref[...]
    o_ref[...] = _dot(x, wout_ref[...]) + bout_ref[...]


def _rows_spec():
    return pl.BlockSpec((_TCB, D), lambda i: (i, 0))


def _w_spec():
    return pl.BlockSpec((D, D), lambda i: (0, 0))


def _b_spec():
    return pl.BlockSpec((1, D), lambda i: (0, 0))


def _pair_spec(last):
    return pl.BlockSpec((2, _TCB, last), lambda i: (0, i, 0))


_tc_head = pl.pallas_call(
    _tc_head_body,
    grid=(_TCG,),
    in_specs=[_rows_spec(), _w_spec(), _b_spec(), _w_spec(), _w_spec(),
              _b_spec()],
    out_specs=[_rows_spec(), _rows_spec()],
    out_shape=[jax.ShapeDtypeStruct((N, D), jnp.float32),
               jax.ShapeDtypeStruct((N, D), jnp.float32)],
)

_tc_mid = pl.pallas_call(
    _tc_mid_body,
    grid=(_TCG,),
    in_specs=[_pair_spec(D), _pair_spec(D), _w_spec(), _b_spec(), _w_spec(),
              _w_spec(), _b_spec()],
    out_specs=[_rows_spec(), _rows_spec()],
    out_shape=[jax.ShapeDtypeStruct((N, D), jnp.float32),
               jax.ShapeDtypeStruct((N, D), jnp.float32)],
)

_tc_tail = pl.pallas_call(
    _tc_tail_body,
    grid=(_TCG,),
    in_specs=[_pair_spec(D), _pair_spec(D), _w_spec(), _b_spec(), _w_spec(),
              _b_spec()],
    out_specs=_rows_spec(),
    out_shape=jax.ShapeDtypeStruct((N, D), jnp.float32),
)


def _zero_acc(sid, src_buf, acc_sh):
    for k in range(ZITER):
        idx = sid + NS * k
        @pl.when(idx < NZCHUNK)
        def _():
            rows = pl.ds(pl.multiple_of(idx * ZROWS, 8), ZROWS)
            pltpu.sync_copy(src_buf, acc_sh.at[rows])


def _writeout_acc(cid, sid, acc_sh, out_hbm):
    for k in range(ZITER):
        idx = sid + NS * k
        @pl.when(idx < NZCHUNK)
        def _():
            rows = pl.ds(pl.multiple_of(idx * ZROWS, 8), ZROWS)
            pltpu.sync_copy(acc_sh.at[rows], out_hbm.at[cid].at[rows])


def _sc_edge_body(u_hbm, v_hbm, src_hbm, dst_hbm, out_hbm,
                  acc_sh, srcv, dstv, urows, vrows, sem):
    cid = lax.axis_index("c")
    sid = lax.axis_index("s")
    wid = sid * NC + cid

    zero = jnp.zeros((L,), jnp.float32)

    def zrow(i, _):
        for g in range(G):
            urows[i, pl.ds(g * L, L)] = zero
        return 0
    lax.fori_loop(0, C, zrow, 0)
    _zero_acc(sid, urows, acc_sh)

    plsc.subcore_barrier()

    base0 = wid * EPW

    def chunk(i, _):
        b = pl.multiple_of(base0 + i * C, 8)
        pltpu.sync_copy(src_hbm.at[pl.ds(b, C)], srcv)
        pltpu.sync_copy(dst_hbm.at[pl.ds(b, C)], dstv)
        cp_u = pltpu.async_copy(u_hbm.at[srcv], urows, sem)
        cp_v = pltpu.async_copy(v_hbm.at[dstv], vrows, sem)
        cp_u.wait()
        cp_v.wait()

        def row(r, _):
            for g in range(G):
                a = urows[r, pl.ds(g * L, L)] + vrows[r, pl.ds(g * L, L)]
                urows[r, pl.ds(g * L, L)] = jnp.maximum(a, 0.0)
            return 0
        lax.fori_loop(0, C, row, 0)

        pltpu.sync_copy(urows, acc_sh.at[dstv], add=True)
        return 0

    lax.fori_loop(0, NCHUNK, chunk, 0)

    plsc.subcore_barrier()
    _writeout_acc(cid, sid, acc_sh, out_hbm)


def _sc_deg_body(dst_hbm, out_hbm, acc_sh, dstv, onesv, zbuf, sem):
    del sem
    cid = lax.axis_index("c")
    sid = lax.axis_index("s")
    wid = sid * NC + cid

    zero = jnp.zeros((L,), jnp.float32)
    one = jnp.full((L,), 1.0, jnp.float32)

    def fill(i, _):
        for g in range(G):
            zbuf[i, pl.ds(g * L, L)] = zero
            onesv[i, pl.ds(g * L, L)] = one
        return 0
    lax.fori_loop(0, C, fill, 0)
    _zero_acc(sid, zbuf, acc_sh)

    plsc.subcore_barrier()

    base0 = wid * EPW

    def chunk(i, _):
        b = pl.multiple_of(base0 + i * C, 8)
        pltpu.sync_copy(dst_hbm.at[pl.ds(b, C)], dstv)
        pltpu.sync_copy(onesv, acc_sh.at[dstv], add=True)
        return 0

    lax.fori_loop(0, NCHUNK, chunk, 0)

    plsc.subcore_barrier()
    _writeout_acc(cid, sid, acc_sh, out_hbm)


@functools.lru_cache(maxsize=None)
def _sc_mesh():
    return plsc.VectorSubcoreMesh(core_axis_name="c", subcore_axis_name="s",
                                  num_cores=NC, num_subcores=NS)


@functools.lru_cache(maxsize=None)
def _make_sc_edge():
    return pl.kernel(
        _sc_edge_body,
        out_type=jax.ShapeDtypeStruct((NC, N, D), jnp.float32),
        mesh=_sc_mesh(),
        scratch_types=[
            pltpu.VMEM_SHARED((N, D), jnp.float32),
            pltpu.VMEM((C,), jnp.int32),
            pltpu.VMEM((C,), jnp.int32),
            pltpu.VMEM((C, D), jnp.float32),
            pltpu.VMEM((C, D), jnp.float32),
            pltpu.SemaphoreType.DMA,
        ],
    )


@functools.lru_cache(maxsize=None)
def _make_sc_deg():
    return pl.kernel(
        _sc_deg_body,
        out_type=jax.ShapeDtypeStruct((NC, N, D), jnp.float32),
        mesh=_sc_mesh(),
        scratch_types=[
            pltpu.VMEM_SHARED((N, D), jnp.float32),
            pltpu.VMEM((C,), jnp.int32),
            pltpu.VMEM((C, D), jnp.float32),
            pltpu.VMEM((C, D), jnp.float32),
            pltpu.SemaphoreType.DMA,
        ],
    )


def kernel(inputs, frames, edge_index, W_in, b_in,
           W1a_0, b1a_0, W1b_0, b1b_0,
           W1a_1, b1a_1, W1b_1, b1b_1,
           W_out, b_out):
    del frames
    src = edge_index[0]
    dst = edge_index[1]

    b_in2 = b_in.reshape(1, D)
    ba0 = b1a_0.reshape(1, D)
    bb0 = b1b_0.reshape(1, D)
    ba1 = b1a_1.reshape(1, D)
    bb1 = b1b_1.reshape(1, D)
    bo = b_out.reshape(1, D)

    u0, v0 = _tc_head(inputs, W_in, b_in2, W1a_0[:D], W1a_0[D:], ba0)
    degp = _make_sc_deg()(dst)
    p0 = _make_sc_edge()(u0, v0, src, dst)
    u1, v1 = _tc_mid(p0, degp, W1b_0, bb0, W1a_1[:D], W1a_1[D:], ba1)
    p1 = _make_sc_edge()(u1, v1, src, dst)
    return _tc_tail(p1, degp, W1b_1, bb1, W_out, bo)


# speedup vs baseline: 10.3825x; 2.1374x over previous
import functools

import jax
import jax.numpy as jnp
from jax import lax
from jax.experimental import pallas as pl
from jax.experimental.pallas import tpu as pltpu
from jax.experimental.pallas import tpu_sc as plsc

N = 10000
E = 640000
D = 128

NC = 2
NS = 16
L = 16
NW = NC * NS
EPW = E // NW
C = 40
NCHUNK = EPW // C
KSUP = 50
SUPC = KSUP * C
NSUP = NCHUNK // KSUP
ZROWS = C
NZCHUNK = N // ZROWS
ZITER = -(-NZCHUNK // NS)
G = D // L


_TCB = 1000
_TCG = N // _TCB


def _dot(a, b):
    return jnp.dot(a, b, preferred_element_type=jnp.float32)


def _tc_head_body(x_ref, win_ref, bin_ref, wat_ref, wab_ref, bav_ref,
                  u_ref, v_ref):
    x = _dot(x_ref[...], win_ref[...]) + bin_ref[...]
    u_ref[...] = _dot(x, wat_ref[...])
    v_ref[...] = _dot(x, wab_ref[...]) + bav_ref[...]


def _tc_mid_body(p_ref, deg_ref, wb_ref, bb_ref, wat_ref, wab_ref, bav_ref,
                 u_ref, v_ref):
    s = p_ref[0] + p_ref[1]
    deg = deg_ref[0, :, 0:1] + deg_ref[1, :, 0:1]
    x = _dot(s, wb_ref[...]) + deg * bb_ref[...]
    u_ref[...] = _dot(x, wat_ref[...])
    v_ref[...] = _dot(x, wab_ref[...]) + bav_ref[...]


def _tc_tail_body(p_ref, deg_ref, wb_ref, bb_ref, wout_ref, bout_ref, o_ref):
    s = p_ref[0] + p_ref[1]
    deg = deg_ref[0, :, 0:1] + deg_ref[1, :, 0:1]
    x = _dot(s, wb_ref[...]) + deg * bb_ref[...]
    o_ref[...] = _dot(x, wout_ref[...]) + bout_ref[...]


def _rows_spec():
    return pl.BlockSpec((_TCB, D), lambda i: (i, 0))


def _w_spec():
    return pl.BlockSpec((D, D), lambda i: (0, 0))


def _b_spec():
    return pl.BlockSpec((1, D), lambda i: (0, 0))


def _pair_spec(last):
    return pl.BlockSpec((2, _TCB, last), lambda i: (0, i, 0))


_tc_head = pl.pallas_call(
    _tc_head_body,
    grid=(_TCG,),
    in_specs=[_rows_spec(), _w_spec(), _b_spec(), _w_spec(), _w_spec(),
              _b_spec()],
    out_specs=[_rows_spec(), _rows_spec()],
    out_shape=[jax.ShapeDtypeStruct((N, D), jnp.float32),
               jax.ShapeDtypeStruct((N, D), jnp.float32)],
)

_tc_mid = pl.pallas_call(
    _tc_mid_body,
    grid=(_TCG,),
    in_specs=[_pair_spec(D), _pair_spec(D), _w_spec(), _b_spec(), _w_spec(),
              _w_spec(), _b_spec()],
    out_specs=[_rows_spec(), _rows_spec()],
    out_shape=[jax.ShapeDtypeStruct((N, D), jnp.float32),
               jax.ShapeDtypeStruct((N, D), jnp.float32)],
)

_tc_tail = pl.pallas_call(
    _tc_tail_body,
    grid=(_TCG,),
    in_specs=[_pair_spec(D), _pair_spec(D), _w_spec(), _b_spec(), _w_spec(),
              _b_spec()],
    out_specs=_rows_spec(),
    out_shape=jax.ShapeDtypeStruct((N, D), jnp.float32),
)


def _zero_acc(sid, src_buf, acc_sh):
    for k in range(ZITER):
        idx = sid + NS * k
        @pl.when(idx < NZCHUNK)
        def _():
            rows = pl.ds(pl.multiple_of(idx * ZROWS, 8), ZROWS)
            pltpu.sync_copy(src_buf, acc_sh.at[rows])


def _writeout_acc(cid, sid, acc_sh, out_hbm):
    for k in range(ZITER):
        idx = sid + NS * k
        @pl.when(idx < NZCHUNK)
        def _():
            rows = pl.ds(pl.multiple_of(idx * ZROWS, 8), ZROWS)
            pltpu.sync_copy(acc_sh.at[rows], out_hbm.at[cid].at[rows])


def _sc_edge_body(u_hbm, v_hbm, src_hbm, dst_hbm, out_hbm,
                  acc_sh, src2, dst2, sf0, sf1, df0, df1,
                  u0b, u1b, v0b, v1b, sem0, sem1):
    cid = lax.axis_index("c")
    sid = lax.axis_index("s")
    wid = sid * NC + cid

    zero = jnp.zeros((L,), jnp.float32)

    def zrow(i, _):
        for g in range(G):
            u0b[i, pl.ds(g * L, L)] = zero
        return 0
    lax.fori_loop(0, C, zrow, 0)
    _zero_acc(sid, u0b, acc_sh)

    plsc.subcore_barrier()

    base0 = wid * EPW
    ubufs = (u0b, u1b)
    vbufs = (v0b, v1b)
    sfl = (sf0, sf1)
    dfl = (df0, df1)
    sems = (sem0, sem1)

    def flat_idx(j, b):
        off = j * C
        for g in range(C // L):
            sfl[b][pl.ds(g * L, L)] = src2[pl.ds(off + g * L, L)]
            dfl[b][pl.ds(g * L, L)] = dst2[pl.ds(off + g * L, L)]
        if C % L:
            t = C - L
            sfl[b][pl.ds(t, L)] = src2[pl.ds(off + t, L)]
            dfl[b][pl.ds(t, L)] = dst2[pl.ds(off + t, L)]

    def issue(b):
        pltpu.async_copy(u_hbm.at[sfl[b]], ubufs[b], sems[b])
        pltpu.async_copy(v_hbm.at[dfl[b]], vbufs[b], sems[b])

    def wait(b):
        pltpu.make_async_copy(u_hbm.at[sfl[b]], ubufs[b], sems[b]).wait()
        pltpu.make_async_copy(v_hbm.at[dfl[b]], vbufs[b], sems[b]).wait()

    def sup(s, _):
        sbase = pl.multiple_of(base0 + s * SUPC, 8)
        pltpu.sync_copy(src_hbm.at[pl.ds(sbase, SUPC)], src2)
        pltpu.sync_copy(dst_hbm.at[pl.ds(sbase, SUPC)], dst2)
        for b in range(2):
            flat_idx(b, b)
            issue(b)

        def pair(p, _):
            for b in range(2):
                j = 2 * p + b
                ub = ubufs[b]
                vb = vbufs[b]
                wait(b)

                def row(r, _):
                    for g in range(G):
                        a = ub[r, pl.ds(g * L, L)] + vb[r, pl.ds(g * L, L)]
                        ub[r, pl.ds(g * L, L)] = jnp.maximum(a, 0.0)
                    return 0
                lax.fori_loop(0, C, row, 0)

                pltpu.sync_copy(ub, acc_sh.at[dfl[b]], add=True)

                @pl.when(j + 2 < KSUP)
                def _():
                    flat_idx(j + 2, b)
                    issue(b)
            return 0

        lax.fori_loop(0, KSUP // 2, pair, 0)
        return 0

    lax.fori_loop(0, NSUP, sup, 0)

    plsc.subcore_barrier()
    _writeout_acc(cid, sid, acc_sh, out_hbm)


def _sc_deg_body(dst_hbm, out_hbm, acc_sh, dstv, onesv, zbuf, sem):
    del sem
    cid = lax.axis_index("c")
    sid = lax.axis_index("s")
    wid = sid * NC + cid

    zero = jnp.zeros((L,), jnp.float32)
    one = jnp.full((L,), 1.0, jnp.float32)

    def fill(i, _):
        for g in range(G):
            zbuf[i, pl.ds(g * L, L)] = zero
            onesv[i, pl.ds(g * L, L)] = one
        return 0
    lax.fori_loop(0, C, fill, 0)
    _zero_acc(sid, zbuf, acc_sh)

    plsc.subcore_barrier()

    base0 = wid * EPW

    def chunk(i, _):
        b = pl.multiple_of(base0 + i * C, 8)
        pltpu.sync_copy(dst_hbm.at[pl.ds(b, C)], dstv)
        pltpu.sync_copy(onesv, acc_sh.at[dstv], add=True)
        return 0

    lax.fori_loop(0, NCHUNK, chunk, 0)

    plsc.subcore_barrier()
    _writeout_acc(cid, sid, acc_sh, out_hbm)


@functools.lru_cache(maxsize=None)
def _sc_mesh():
    return plsc.VectorSubcoreMesh(core_axis_name="c", subcore_axis_name="s",
                                  num_cores=NC, num_subcores=NS)


@functools.lru_cache(maxsize=None)
def _make_sc_edge():
    return pl.kernel(
        _sc_edge_body,
        out_type=jax.ShapeDtypeStruct((NC, N, D), jnp.float32),
        mesh=_sc_mesh(),
        scratch_types=[
            pltpu.VMEM_SHARED((N, D), jnp.float32),
            pltpu.VMEM((SUPC,), jnp.int32),
            pltpu.VMEM((SUPC,), jnp.int32),
            pltpu.VMEM((C,), jnp.int32),
            pltpu.VMEM((C,), jnp.int32),
            pltpu.VMEM((C,), jnp.int32),
            pltpu.VMEM((C,), jnp.int32),
            pltpu.VMEM((C, D), jnp.float32),
            pltpu.VMEM((C, D), jnp.float32),
            pltpu.VMEM((C, D), jnp.float32),
            pltpu.VMEM((C, D), jnp.float32),
            pltpu.SemaphoreType.DMA,
            pltpu.SemaphoreType.DMA,
        ],
    )


@functools.lru_cache(maxsize=None)
def _make_sc_deg():
    return pl.kernel(
        _sc_deg_body,
        out_type=jax.ShapeDtypeStruct((NC, N, D), jnp.float32),
        mesh=_sc_mesh(),
        scratch_types=[
            pltpu.VMEM_SHARED((N, D), jnp.float32),
            pltpu.VMEM((C,), jnp.int32),
            pltpu.VMEM((C, D), jnp.float32),
            pltpu.VMEM((C, D), jnp.float32),
            pltpu.SemaphoreType.DMA,
        ],
    )


def kernel(inputs, frames, edge_index, W_in, b_in,
           W1a_0, b1a_0, W1b_0, b1b_0,
           W1a_1, b1a_1, W1b_1, b1b_1,
           W_out, b_out):
    del frames
    src = edge_index[0]
    dst = edge_index[1]

    b_in2 = b_in.reshape(1, D)
    ba0 = b1a_0.reshape(1, D)
    bb0 = b1b_0.reshape(1, D)
    ba1 = b1a_1.reshape(1, D)
    bb1 = b1b_1.reshape(1, D)
    bo = b_out.reshape(1, D)

    u0, v0 = _tc_head(inputs, W_in, b_in2, W1a_0[:D], W1a_0[D:], ba0)
    degp = _make_sc_deg()(dst)
    p0 = _make_sc_edge()(u0, v0, src, dst)
    u1, v1 = _tc_mid(p0, degp, W1b_0, bb0, W1a_1[:D], W1a_1[D:], ba1)
    p1 = _make_sc_edge()(u1, v1, src, dst)
    return _tc_tail(p1, degp, W1b_1, bb1, W_out, bo)


# speedup vs baseline: 13.3076x; 1.2817x over previous
import functools

import jax
import jax.numpy as jnp
from jax import lax
from jax.experimental import pallas as pl
from jax.experimental.pallas import tpu as pltpu
from jax.experimental.pallas import tpu_sc as plsc

N = 10000
E = 640000
D = 128

NC = 2
NS = 16
L = 16
NW = NC * NS
EPW = E // NW
C = 40
NCHUNK = EPW // C
KSUP = 50
SUPC = KSUP * C
NSUP = NCHUNK // KSUP
ZROWS = C
NZCHUNK = N // ZROWS
ZITER = -(-NZCHUNK // NS)
G = D // L


_TCB = 1000
_TCG = N // _TCB


def _dot(a, b):
    return jnp.dot(a, b, preferred_element_type=jnp.float32)


def _tc_head_body(x_ref, win_ref, bin_ref, wat_ref, wab_ref, bav_ref,
                  u_ref, v_ref):
    x = _dot(x_ref[...], win_ref[...]) + bin_ref[...]
    u_ref[...] = _dot(x, wat_ref[...])
    v_ref[...] = _dot(x, wab_ref[...]) + bav_ref[...]


def _tc_mid_body(p_ref, deg_ref, wb_ref, bb_ref, wat_ref, wab_ref, bav_ref,
                 u_ref, v_ref):
    s = p_ref[0] + p_ref[1]
    deg = deg_ref[0, :, 0:1] + deg_ref[1, :, 0:1]
    x = _dot(s, wb_ref[...]) + deg * bb_ref[...]
    u_ref[...] = _dot(x, wat_ref[...])
    v_ref[...] = _dot(x, wab_ref[...]) + bav_ref[...]


def _tc_tail_body(p_ref, deg_ref, wb_ref, bb_ref, wout_ref, bout_ref, o_ref):
    s = p_ref[0] + p_ref[1]
    deg = deg_ref[0, :, 0:1] + deg_ref[1, :, 0:1]
    x = _dot(s, wb_ref[...]) + deg * bb_ref[...]
    o_ref[...] = _dot(x, wout_ref[...]) + bout_ref[...]


def _rows_spec():
    return pl.BlockSpec((_TCB, D), lambda i: (i, 0))


def _w_spec():
    return pl.BlockSpec((D, D), lambda i: (0, 0))


def _b_spec():
    return pl.BlockSpec((1, D), lambda i: (0, 0))


def _pair_spec(last):
    return pl.BlockSpec((2, _TCB, last), lambda i: (0, i, 0))


_tc_head = pl.pallas_call(
    _tc_head_body,
    grid=(_TCG,),
    in_specs=[_rows_spec(), _w_spec(), _b_spec(), _w_spec(), _w_spec(),
              _b_spec()],
    out_specs=[_rows_spec(), _rows_spec()],
    out_shape=[jax.ShapeDtypeStruct((N, D), jnp.float32),
               jax.ShapeDtypeStruct((N, D), jnp.float32)],
)

_tc_mid = pl.pallas_call(
    _tc_mid_body,
    grid=(_TCG,),
    in_specs=[_pair_spec(D), _pair_spec(D), _w_spec(), _b_spec(), _w_spec(),
              _w_spec(), _b_spec()],
    out_specs=[_rows_spec(), _rows_spec()],
    out_shape=[jax.ShapeDtypeStruct((N, D), jnp.float32),
               jax.ShapeDtypeStruct((N, D), jnp.float32)],
)

_tc_tail = pl.pallas_call(
    _tc_tail_body,
    grid=(_TCG,),
    in_specs=[_pair_spec(D), _pair_spec(D), _w_spec(), _b_spec(), _w_spec(),
              _b_spec()],
    out_specs=_rows_spec(),
    out_shape=jax.ShapeDtypeStruct((N, D), jnp.float32),
)


def _zero_acc(sid, src_buf, acc_sh):
    for k in range(ZITER):
        idx = sid + NS * k
        @pl.when(idx < NZCHUNK)
        def _():
            rows = pl.ds(pl.multiple_of(idx * ZROWS, 8), ZROWS)
            pltpu.sync_copy(src_buf, acc_sh.at[rows])


def _writeout_acc(cid, sid, acc_sh, out_hbm):
    for k in range(ZITER):
        idx = sid + NS * k
        @pl.when(idx < NZCHUNK)
        def _():
            rows = pl.ds(pl.multiple_of(idx * ZROWS, 8), ZROWS)
            pltpu.sync_copy(acc_sh.at[rows], out_hbm.at[cid].at[rows])


def _sc_edge_body(u_hbm, v_hbm, src_hbm, dst_hbm, out_hbm,
                  acc_sh, src2, dst2, sf0, sf1, df0, df1, dc0, dc1,
                  u0b, u1b, v0b, v1b, s0b, s1b, sem0, sem1):
    cid = lax.axis_index("c")
    sid = lax.axis_index("s")
    wid = sid * NC + cid

    zero = jnp.zeros((L,), jnp.float32)

    def zrow(i, _):
        for g in range(G):
            u0b[i, pl.ds(g * L, L)] = zero
        return 0
    lax.fori_loop(0, C, zrow, 0)
    _zero_acc(sid, u0b, acc_sh)

    plsc.subcore_barrier()

    base0 = wid * EPW
    ubufs = (u0b, u1b)
    vbufs = (v0b, v1b)
    sbufs = (s0b, s1b)
    sfl = (sf0, sf1)
    dfl = (df0, df1)
    dscs = (dc0, dc1)
    sems = (sem0, sem1)

    def flat_idx(j, b):
        off = j * C
        for g in range(C // L):
            sfl[b][pl.ds(g * L, L)] = src2[pl.ds(off + g * L, L)]
            dfl[b][pl.ds(g * L, L)] = dst2[pl.ds(off + g * L, L)]
        if C % L:
            t = C - L
            sfl[b][pl.ds(t, L)] = src2[pl.ds(off + t, L)]
            dfl[b][pl.ds(t, L)] = dst2[pl.ds(off + t, L)]

    def issue(b):
        pltpu.async_copy(u_hbm.at[sfl[b]], ubufs[b], sems[b])
        pltpu.async_copy(v_hbm.at[dfl[b]], vbufs[b], sems[b])

    def wait(b):
        pltpu.make_async_copy(u_hbm.at[sfl[b]], ubufs[b], sems[b]).wait()
        pltpu.make_async_copy(v_hbm.at[dfl[b]], vbufs[b], sems[b]).wait()

    def sup(s, _):
        sbase = pl.multiple_of(base0 + s * SUPC, 8)
        pltpu.sync_copy(src_hbm.at[pl.ds(sbase, SUPC)], src2)
        pltpu.sync_copy(dst_hbm.at[pl.ds(sbase, SUPC)], dst2)
        for b in range(2):
            flat_idx(b, b)
            issue(b)

        def pair(p, _):
            for b in range(2):
                j = 2 * p + b
                ub = ubufs[b]
                vb = vbufs[b]
                sb = sbufs[b]
                db = dscs[b]
                wait(b)
                for g in range(C // L):
                    db[pl.ds(g * L, L)] = dfl[b][pl.ds(g * L, L)]
                if C % L:
                    t = C - L
                    db[pl.ds(t, L)] = dfl[b][pl.ds(t, L)]

                def row(r, _):
                    for g in range(G):
                        a = ub[r, pl.ds(g * L, L)] + vb[r, pl.ds(g * L, L)]
                        sb[r, pl.ds(g * L, L)] = jnp.maximum(a, 0.0)
                    return 0
                lax.fori_loop(0, C, row, 0)

                @pl.when(j + 2 < KSUP)
                def _():
                    flat_idx(j + 2, b)
                    issue(b)

                pltpu.sync_copy(sb, acc_sh.at[db], add=True)
            return 0

        lax.fori_loop(0, KSUP // 2, pair, 0)
        return 0

    lax.fori_loop(0, NSUP, sup, 0)

    plsc.subcore_barrier()
    _writeout_acc(cid, sid, acc_sh, out_hbm)


def _sc_deg_body(dst_hbm, out_hbm, acc_sh, dst2, dstv, onesv, zbuf):
    cid = lax.axis_index("c")
    sid = lax.axis_index("s")
    wid = sid * NC + cid

    zero = jnp.zeros((L,), jnp.float32)
    one = jnp.full((L,), 1.0, jnp.float32)

    def fill(i, _):
        for g in range(G):
            zbuf[i, pl.ds(g * L, L)] = zero
            onesv[i, pl.ds(g * L, L)] = one
        return 0
    lax.fori_loop(0, C, fill, 0)
    _zero_acc(sid, zbuf, acc_sh)

    plsc.subcore_barrier()

    base0 = wid * EPW

    def sup(s, _):
        sbase = pl.multiple_of(base0 + s * SUPC, 8)
        pltpu.sync_copy(dst_hbm.at[pl.ds(sbase, SUPC)], dst2)

        def chunk(j, _):
            off = j * C
            for g in range(C // L):
                dstv[pl.ds(g * L, L)] = dst2[pl.ds(off + g * L, L)]
            if C % L:
                t = C - L
                dstv[pl.ds(t, L)] = dst2[pl.ds(off + t, L)]
            pltpu.sync_copy(onesv, acc_sh.at[dstv], add=True)
            return 0

        lax.fori_loop(0, KSUP, chunk, 0)
        return 0

    lax.fori_loop(0, NSUP, sup, 0)

    plsc.subcore_barrier()
    _writeout_acc(cid, sid, acc_sh, out_hbm)


@functools.lru_cache(maxsize=None)
def _sc_mesh():
    return plsc.VectorSubcoreMesh(core_axis_name="c", subcore_axis_name="s",
                                  num_cores=NC, num_subcores=NS)


@functools.lru_cache(maxsize=None)
def _make_sc_edge():
    return pl.kernel(
        _sc_edge_body,
        out_type=jax.ShapeDtypeStruct((NC, N, D), jnp.float32),
        mesh=_sc_mesh(),
        scratch_types=[
            pltpu.VMEM_SHARED((N, D), jnp.float32),
            pltpu.VMEM((SUPC,), jnp.int32),
            pltpu.VMEM((SUPC,), jnp.int32),
            pltpu.VMEM((C,), jnp.int32),
            pltpu.VMEM((C,), jnp.int32),
            pltpu.VMEM((C,), jnp.int32),
            pltpu.VMEM((C,), jnp.int32),
            pltpu.VMEM((C,), jnp.int32),
            pltpu.VMEM((C,), jnp.int32),
            pltpu.VMEM((C, D), jnp.float32),
            pltpu.VMEM((C, D), jnp.float32),
            pltpu.VMEM((C, D), jnp.float32),
            pltpu.VMEM((C, D), jnp.float32),
            pltpu.VMEM((C, D), jnp.float32),
            pltpu.VMEM((C, D), jnp.float32),
            pltpu.SemaphoreType.DMA,
            pltpu.SemaphoreType.DMA,
        ],
    )


@functools.lru_cache(maxsize=None)
def _make_sc_deg():
    return pl.kernel(
        _sc_deg_body,
        out_type=jax.ShapeDtypeStruct((NC, N, D), jnp.float32),
        mesh=_sc_mesh(),
        scratch_types=[
            pltpu.VMEM_SHARED((N, D), jnp.float32),
            pltpu.VMEM((SUPC,), jnp.int32),
            pltpu.VMEM((C,), jnp.int32),
            pltpu.VMEM((C, D), jnp.float32),
            pltpu.VMEM((C, D), jnp.float32),
        ],
    )


def kernel(inputs, frames, edge_index, W_in, b_in,
           W1a_0, b1a_0, W1b_0, b1b_0,
           W1a_1, b1a_1, W1b_1, b1b_1,
           W_out, b_out):
    del frames
    src = edge_index[0]
    dst = edge_index[1]

    b_in2 = b_in.reshape(1, D)
    ba0 = b1a_0.reshape(1, D)
    bb0 = b1b_0.reshape(1, D)
    ba1 = b1a_1.reshape(1, D)
    bb1 = b1b_1.reshape(1, D)
    bo = b_out.reshape(1, D)

    u0, v0 = _tc_head(inputs, W_in, b_in2, W1a_0[:D], W1a_0[D:], ba0)
    degp = _make_sc_deg()(dst)
    p0 = _make_sc_edge()(u0, v0, src, dst)
    u1, v1 = _tc_mid(p0, degp, W1b_0, bb0, W1a_1[:D], W1a_1[D:], ba1)
    p1 = _make_sc_edge()(u1, v1, src, dst)
    return _tc_tail(p1, degp, W1b_1, bb1, W_out, bo)


# speedup vs baseline: 13.5606x; 1.0190x over previous
import functools

import jax
import jax.numpy as jnp
from jax import lax
from jax.experimental import pallas as pl
from jax.experimental.pallas import tpu as pltpu
from jax.experimental.pallas import tpu_sc as plsc

N = 10000
E = 640000
D = 128

NC = 2
NS = 16
L = 16
NW = NC * NS
EPW = E // NW
C = 40
NCHUNK = EPW // C
KSUP = 50
SUPC = KSUP * C
NSUP = NCHUNK // KSUP
ZROWS = C
NZCHUNK = N // ZROWS
ZITER = -(-NZCHUNK // NS)
G = D // L


_TCB = 1000
_TCG = N // _TCB


def _dot(a, b):
    return jnp.dot(a, b, preferred_element_type=jnp.float32)


def _tc_head_body(x_ref, win_ref, bin_ref, wat_ref, wab_ref, bav_ref,
                  u_ref, v_ref):
    x = _dot(x_ref[...], win_ref[...]) + bin_ref[...]
    u_ref[...] = _dot(x, wat_ref[...])
    v_ref[...] = _dot(x, wab_ref[...]) + bav_ref[...]


def _tc_mid_body(p_ref, deg_ref, wb_ref, bb_ref, wat_ref, wab_ref, bav_ref,
                 u_ref, v_ref):
    s = p_ref[0] + p_ref[1]
    deg = deg_ref[0, :, 0:1] + deg_ref[1, :, 0:1]
    x = _dot(s, wb_ref[...]) + deg * bb_ref[...]
    u_ref[...] = _dot(x, wat_ref[...])
    v_ref[...] = _dot(x, wab_ref[...]) + bav_ref[...]


def _tc_tail_body(p_ref, deg_ref, wb_ref, bb_ref, wout_ref, bout_ref, o_ref):
    s = p_ref[0] + p_ref[1]
    deg = deg_ref[0, :, 0:1] + deg_ref[1, :, 0:1]
    x = _dot(s, wb_ref[...]) + deg * bb_ref[...]
    o_ref[...] = _dot(x, wout_ref[...]) + bout_ref[...]


def _rows_spec():
    return pl.BlockSpec((_TCB, D), lambda i: (i, 0))


def _w_spec():
    return pl.BlockSpec((D, D), lambda i: (0, 0))


def _b_spec():
    return pl.BlockSpec((1, D), lambda i: (0, 0))


def _pair_spec(last):
    return pl.BlockSpec((2, _TCB, last), lambda i: (0, i, 0))


_tc_head = pl.pallas_call(
    _tc_head_body,
    grid=(_TCG,),
    in_specs=[_rows_spec(), _w_spec(), _b_spec(), _w_spec(), _w_spec(),
              _b_spec()],
    out_specs=[_rows_spec(), _rows_spec()],
    out_shape=[jax.ShapeDtypeStruct((N, D), jnp.float32),
               jax.ShapeDtypeStruct((N, D), jnp.float32)],
)

_tc_mid = pl.pallas_call(
    _tc_mid_body,
    grid=(_TCG,),
    in_specs=[_pair_spec(D), _pair_spec(D), _w_spec(), _b_spec(), _w_spec(),
              _w_spec(), _b_spec()],
    out_specs=[_rows_spec(), _rows_spec()],
    out_shape=[jax.ShapeDtypeStruct((N, D), jnp.float32),
               jax.ShapeDtypeStruct((N, D), jnp.float32)],
)

_tc_tail = pl.pallas_call(
    _tc_tail_body,
    grid=(_TCG,),
    in_specs=[_pair_spec(D), _pair_spec(D), _w_spec(), _b_spec(), _w_spec(),
              _b_spec()],
    out_specs=_rows_spec(),
    out_shape=jax.ShapeDtypeStruct((N, D), jnp.float32),
)


def _zero_acc(sid, src_buf, acc_sh):
    for k in range(ZITER):
        idx = sid + NS * k
        @pl.when(idx < NZCHUNK)
        def _():
            rows = pl.ds(pl.multiple_of(idx * ZROWS, 8), ZROWS)
            pltpu.sync_copy(src_buf, acc_sh.at[rows])


def _writeout_acc(cid, sid, acc_sh, out_hbm):
    for k in range(ZITER):
        idx = sid + NS * k
        @pl.when(idx < NZCHUNK)
        def _():
            rows = pl.ds(pl.multiple_of(idx * ZROWS, 8), ZROWS)
            pltpu.sync_copy(acc_sh.at[rows], out_hbm.at[cid].at[rows])


def _sc_edge_body(u_hbm, v_hbm, src_hbm, dst_hbm, out_hbm,
                  acc_sh, src2, dst2, sf0, sf1, df0, df1, dc0, dc1,
                  u0b, u1b, v0b, v1b, s0b, s1b, sem0, sem1, ssem0, ssem1):
    cid = lax.axis_index("c")
    sid = lax.axis_index("s")
    wid = sid * NC + cid

    zero = jnp.zeros((L,), jnp.float32)

    def zrow(i, _):
        for g in range(G):
            u0b[i, pl.ds(g * L, L)] = zero
        return 0
    lax.fori_loop(0, C, zrow, 0)
    _zero_acc(sid, u0b, acc_sh)

    plsc.subcore_barrier()

    base0 = wid * EPW
    ubufs = (u0b, u1b)
    vbufs = (v0b, v1b)
    sbufs = (s0b, s1b)
    sfl = (sf0, sf1)
    dfl = (df0, df1)
    dscs = (dc0, dc1)
    sems = (sem0, sem1)
    ssems = (ssem0, ssem1)

    def flat_idx(j, b):
        off = j * C
        for g in range(C // L):
            sfl[b][pl.ds(g * L, L)] = src2[pl.ds(off + g * L, L)]
            dfl[b][pl.ds(g * L, L)] = dst2[pl.ds(off + g * L, L)]
        if C % L:
            t = C - L
            sfl[b][pl.ds(t, L)] = src2[pl.ds(off + t, L)]
            dfl[b][pl.ds(t, L)] = dst2[pl.ds(off + t, L)]

    def issue(b):
        pltpu.async_copy(u_hbm.at[sfl[b]], ubufs[b], sems[b])
        pltpu.async_copy(v_hbm.at[dfl[b]], vbufs[b], sems[b])

    def wait(b):
        pltpu.make_async_copy(u_hbm.at[sfl[b]], ubufs[b], sems[b]).wait()
        pltpu.make_async_copy(v_hbm.at[dfl[b]], vbufs[b], sems[b]).wait()

    def sup(s, _):
        sbase = pl.multiple_of(base0 + s * SUPC, 8)
        pltpu.sync_copy(src_hbm.at[pl.ds(sbase, SUPC)], src2)
        pltpu.sync_copy(dst_hbm.at[pl.ds(sbase, SUPC)], dst2)
        for b in range(2):
            flat_idx(b, b)
            issue(b)

        def pair(p, _):
            for b in range(2):
                j = 2 * p + b
                ub = ubufs[b]
                vb = vbufs[b]
                sb = sbufs[b]
                db = dscs[b]
                wait(b)
                @pl.when(j >= 2)
                def _():
                    pltpu.make_async_copy(sb, acc_sh.at[db], ssems[b]).wait()

                for g in range(C // L):
                    db[pl.ds(g * L, L)] = dfl[b][pl.ds(g * L, L)]
                if C % L:
                    t = C - L
                    db[pl.ds(t, L)] = dfl[b][pl.ds(t, L)]

                def row(r, _):
                    for g in range(G):
                        a = ub[r, pl.ds(g * L, L)] + vb[r, pl.ds(g * L, L)]
                        sb[r, pl.ds(g * L, L)] = jnp.maximum(a, 0.0)
                    return 0
                lax.fori_loop(0, C, row, 0)

                @pl.when(j + 2 < KSUP)
                def _():
                    flat_idx(j + 2, b)
                    issue(b)

                pltpu.async_copy(sb, acc_sh.at[db], ssems[b], add=True)
            return 0

        lax.fori_loop(0, KSUP // 2, pair, 0)
        for b in range(2):
            pltpu.make_async_copy(sbufs[b], acc_sh.at[dscs[b]],
                                  ssems[b]).wait()
        return 0

    lax.fori_loop(0, NSUP, sup, 0)

    plsc.subcore_barrier()
    _writeout_acc(cid, sid, acc_sh, out_hbm)


def _sc_deg_body(dst_hbm, out_hbm, acc_sh, dst2, df0, df1, onesv, zbuf,
                 sem0, sem1):
    cid = lax.axis_index("c")
    sid = lax.axis_index("s")
    wid = sid * NC + cid

    zero = jnp.zeros((L,), jnp.float32)
    one = jnp.full((L,), 1.0, jnp.float32)

    def fill(i, _):
        for g in range(G):
            zbuf[i, pl.ds(g * L, L)] = zero
            onesv[i, pl.ds(g * L, L)] = one
        return 0
    lax.fori_loop(0, C, fill, 0)
    _zero_acc(sid, zbuf, acc_sh)

    plsc.subcore_barrier()

    base0 = wid * EPW
    dfl = (df0, df1)
    sems = (sem0, sem1)

    def sup(s, _):
        sbase = pl.multiple_of(base0 + s * SUPC, 8)
        pltpu.sync_copy(dst_hbm.at[pl.ds(sbase, SUPC)], dst2)

        def pair(p, _):
            for b in range(2):
                j = 2 * p + b
                db = dfl[b]
                @pl.when(j >= 2)
                def _():
                    pltpu.make_async_copy(onesv, acc_sh.at[db],
                                          sems[b]).wait()
                off = j * C
                for g in range(C // L):
                    db[pl.ds(g * L, L)] = dst2[pl.ds(off + g * L, L)]
                if C % L:
                    t = C - L
                    db[pl.ds(t, L)] = dst2[pl.ds(off + t, L)]
                pltpu.async_copy(onesv, acc_sh.at[db], sems[b], add=True)
            return 0

        lax.fori_loop(0, KSUP // 2, pair, 0)
        for b in range(2):
            pltpu.make_async_copy(onesv, acc_sh.at[dfl[b]], sems[b]).wait()
        return 0

    lax.fori_loop(0, NSUP, sup, 0)

    plsc.subcore_barrier()
    _writeout_acc(cid, sid, acc_sh, out_hbm)


@functools.lru_cache(maxsize=None)
def _sc_mesh():
    return plsc.VectorSubcoreMesh(core_axis_name="c", subcore_axis_name="s",
                                  num_cores=NC, num_subcores=NS)


@functools.lru_cache(maxsize=None)
def _make_sc_edge():
    return pl.kernel(
        _sc_edge_body,
        out_type=jax.ShapeDtypeStruct((NC, N, D), jnp.float32),
        mesh=_sc_mesh(),
        scratch_types=[
            pltpu.VMEM_SHARED((N, D), jnp.float32),
            pltpu.VMEM((SUPC,), jnp.int32),
            pltpu.VMEM((SUPC,), jnp.int32),
            pltpu.VMEM((C,), jnp.int32),
            pltpu.VMEM((C,), jnp.int32),
            pltpu.VMEM((C,), jnp.int32),
            pltpu.VMEM((C,), jnp.int32),
            pltpu.VMEM((C,), jnp.int32),
            pltpu.VMEM((C,), jnp.int32),
            pltpu.VMEM((C, D), jnp.float32),
            pltpu.VMEM((C, D), jnp.float32),
            pltpu.VMEM((C, D), jnp.float32),
            pltpu.VMEM((C, D), jnp.float32),
            pltpu.VMEM((C, D), jnp.float32),
            pltpu.VMEM((C, D), jnp.float32),
            pltpu.SemaphoreType.DMA,
            pltpu.SemaphoreType.DMA,
            pltpu.SemaphoreType.DMA,
            pltpu.SemaphoreType.DMA,
        ],
    )


@functools.lru_cache(maxsize=None)
def _make_sc_deg():
    return pl.kernel(
        _sc_deg_body,
        out_type=jax.ShapeDtypeStruct((NC, N, D), jnp.float32),
        mesh=_sc_mesh(),
        scratch_types=[
            pltpu.VMEM_SHARED((N, D), jnp.float32),
            pltpu.VMEM((SUPC,), jnp.int32),
            pltpu.VMEM((C,), jnp.int32),
            pltpu.VMEM((C,), jnp.int32),
            pltpu.VMEM((C, D), jnp.float32),
            pltpu.VMEM((C, D), jnp.float32),
            pltpu.SemaphoreType.DMA,
            pltpu.SemaphoreType.DMA,
        ],
    )


def kernel(inputs, frames, edge_index, W_in, b_in,
           W1a_0, b1a_0, W1b_0, b1b_0,
           W1a_1, b1a_1, W1b_1, b1b_1,
           W_out, b_out):
    del frames
    src = edge_index[0]
    dst = edge_index[1]

    b_in2 = b_in.reshape(1, D)
    ba0 = b1a_0.reshape(1, D)
    bb0 = b1b_0.reshape(1, D)
    ba1 = b1a_1.reshape(1, D)
    bb1 = b1b_1.reshape(1, D)
    bo = b_out.reshape(1, D)

    u0, v0 = _tc_head(inputs, W_in, b_in2, W1a_0[:D], W1a_0[D:], ba0)
    degp = _make_sc_deg()(dst)
    p0 = _make_sc_edge()(u0, v0, src, dst)
    u1, v1 = _tc_mid(p0, degp, W1b_0, bb0, W1a_1[:D], W1a_1[D:], ba1)
    p1 = _make_sc_edge()(u1, v1, src, dst)
    return _tc_tail(p1, degp, W1b_1, bb1, W_out, bo)
